# baseline (speedup 1.0000x reference)
import functools

import jax
import jax.numpy as jnp
from jax import lax
from jax.experimental import pallas as pl
from jax.experimental.pallas import tpu as pltpu
from jax.experimental.pallas import tpu_sc as plsc

NPOINT = 512
K = 32
B = 16
N = 2048
M = B * NPOINT * K
MB = 4096
QB = MB // K
EPS = 1e-5
NW = 32
RPW = M // NW
QPW = B * NPOINT // NW


def _idx_points(points, idx):
    return jax.vmap(lambda p, i: p[i])(points, idx)


def _fps_sc_body(xyz_hbm, far0_hbm, out_hbm, xyz_v, dist_v, cent_v, far_v):
    wid = lax.axis_index("c") * 16 + lax.axis_index("s")

    @pl.when(wid < B)
    def _():
        pltpu.sync_copy(xyz_hbm.at[wid], xyz_v)
        pltpu.sync_copy(far0_hbm, far_v)
        nchunks = N // 16
        lanes = lax.iota(jnp.int32, 16)

        def init_chunk(c, _):
            dist_v[pl.ds(c * 16, 16)] = jnp.full((16,), 1e10, jnp.float32)
            return 0

        lax.fori_loop(0, nchunks, init_chunk, 0, unroll=4)

        row0 = jnp.zeros((16,), jnp.int32)
        row1 = jnp.ones((16,), jnp.int32)
        row2 = jnp.full((16,), 2, jnp.int32)

        def step(i, carry):
            far_b, cvec = carry
            cvec = jnp.where(lanes == (i % 16), far_b, cvec)

            @pl.when((i % 16) == 15)
            def _():
                cent_v[pl.ds(pl.multiple_of(i - 15, 16), 16)] = cvec

            cx = plsc.load_gather(xyz_v, [row0, far_b])
            cy = plsc.load_gather(xyz_v, [row1, far_b])
            cz = plsc.load_gather(xyz_v, [row2, far_b])

            def chunk4(c4, carry):
                out = []
                for u in range(4):
                    bv, bi = carry[2 * u], carry[2 * u + 1]
                    base = c4 * 64 + u * 16
                    dx = xyz_v[0, pl.ds(base, 16)] - cx
                    dy = xyz_v[1, pl.ds(base, 16)] - cy
                    dz = xyz_v[2, pl.ds(base, 16)] - cz
                    d = dx * dx + dy * dy + dz * dz
                    dmin = jnp.minimum(dist_v[pl.ds(base, 16)], d)
                    dist_v[pl.ds(base, 16)] = dmin
                    m = dmin > bv
                    out.append(jnp.where(m, dmin, bv))
                    out.append(jnp.where(m, base + lanes, bi))
                return tuple(out)

            bv0 = jnp.full((16,), -1.0, jnp.float32)
            bi0 = jnp.zeros((16,), jnp.int32)
            acc = lax.fori_loop(0, nchunks // 4, chunk4,
                                (bv0, bi0) * 4, unroll=2)

            def comb(a, b):
                av, ai = a
                bv_, bi_ = b
                m = (bv_ > av) | ((bv_ == av) & (bi_ < ai))
                return jnp.where(m, bv_, av), jnp.where(m, bi_, ai)

            bv, bi = comb(comb((acc[0], acc[1]), (acc[2], acc[3])),
                          comb((acc[4], acc[5]), (acc[6], acc[7])))
            maxv = jnp.max(bv)
            cand = jnp.where(bv == maxv, bi, jnp.int32(1 << 30))
            far_new = jnp.full((16,), jnp.min(cand), jnp.int32)
            return far_new, cvec

        far_init = plsc.load_gather(far_v, [jnp.full((16,), wid, jnp.int32)])
        lax.fori_loop(0, NPOINT, step, (far_init, jnp.zeros((16,), jnp.int32)))
        pltpu.sync_copy(cent_v, out_hbm.at[wid])


def _fps_sc(xyz_t, far0):
    mesh = plsc.VectorSubcoreMesh(core_axis_name="c", subcore_axis_name="s")
    return pl.kernel(
        _fps_sc_body,
        mesh=mesh,
        compiler_params=pltpu.CompilerParams(needs_layout_passes=False),
        out_type=jax.ShapeDtypeStruct((B, NPOINT), jnp.int32),
        scratch_types=[
            pltpu.VMEM((3, N + 16), jnp.float32),
            pltpu.VMEM((N,), jnp.float32),
            pltpu.VMEM((NPOINT,), jnp.int32),
            pltpu.VMEM((32,), jnp.int32),
        ],
    )(xyz_t, far0)


def _knn_sc_body(xyz_hbm, fps_hbm, out_hbm, xyz_v, fps_v, dbuf, cidx, obuf):
    w = lax.axis_index("c") * 16 + lax.axis_index("s")
    b = w // 2
    qbase = (w % 2) * QPW
    lanes = lax.iota(jnp.int32, 16)
    inf16 = jnp.full((16,), jnp.inf, jnp.float32)
    row0 = jnp.zeros((16,), jnp.int32)
    row1 = jnp.ones((16,), jnp.int32)
    row2 = jnp.full((16,), 2, jnp.int32)

    pltpu.sync_copy(xyz_hbm.at[b], xyz_v)
    pltpu.sync_copy(fps_hbm.at[b], fps_v)
    dbuf[pl.ds(N, 16)] = inf16

    def do_row(q, _):
        qi = qbase + q
        farb = plsc.load_gather(fps_v, [jnp.full((16,), qi, jnp.int32)])
        cx = plsc.load_gather(xyz_v, [row0, farb])
        cy = plsc.load_gather(xyz_v, [row1, farb])
        cz = plsc.load_gather(xyz_v, [row2, farb])

        def dchunk4(c4, hs):
            out = []
            for u in range(4):
                base = c4 * 64 + u * 16
                dx = xyz_v[0, pl.ds(base, 16)] - cx
                dy = xyz_v[1, pl.ds(base, 16)] - cy
                dz = xyz_v[2, pl.ds(base, 16)] - cz
                d = dx * dx + dy * dy + dz * dz
                dbuf[pl.ds(base, 16)] = d
                out.append(jnp.minimum(hs[u], d))
            return tuple(out)

        hs1 = lax.fori_loop(0, 16, dchunk4, (inf16,) * 4)
        hs2 = lax.fori_loop(16, 32, dchunk4, (inf16,) * 4)
        h1 = jnp.minimum(jnp.minimum(hs1[0], hs1[1]), jnp.minimum(hs1[2], hs1[3]))
        h2 = jnp.minimum(jnp.minimum(hs2[0], hs2[1]), jnp.minimum(hs2[2], hs2[3]))
        thr = jnp.maximum(jnp.max(h1), jnp.max(h2))

        def cchunk(c, off):
            base = c * 16
            d = dbuf[pl.ds(base, 16)]
            m = d <= thr
            cs = plsc.cumsum(m.astype(jnp.int32))
            pos = off + cs - 1
            plsc.store_scatter(cidx, [pos], base + lanes, mask=m)
            return off + plsc.all_reduce_population_count(m)[0]

        off = lax.fori_loop(0, 128, cchunk, jnp.int32(0), unroll=4)
        plsc.store_scatter(cidx, [off + lanes], jnp.full((16,), N, jnp.int32))
        nch = (off + 15) // 16

        big16 = jnp.full((16,), 1 << 30, jnp.int32)

        def merge_chunk(c, carry):
            s0k, s0v, s1k, s1v = carry
            idxs = cidx[pl.ds(pl.multiple_of(c * 16, 16), 16)]
            vals = plsc.load_gather(dbuf, [idxs])
            ck, cv = plsc.sort_key_val(vals, idxs)
            rk = lax.rev(ck, (0,))
            rv = lax.rev(cv, (0,))
            m = s1k <= rk
            lok = jnp.where(m, s1k, rk)
            lov = jnp.where(m, s1v, rv)
            lok, lov = plsc.sort_key_val(lok, lov)
            rk = lax.rev(lok, (0,))
            rv = lax.rev(lov, (0,))
            m = s0k <= rk
            n0k = jnp.where(m, s0k, rk)
            n0v = jnp.where(m, s0v, rv)
            n1k = jnp.where(m, rk, s0k)
            n1v = jnp.where(m, rv, s0v)
            n0k, n0v = plsc.sort_key_val(n0k, n0v)
            n1k, n1v = plsc.sort_key_val(n1k, n1v)
            return n0k, n0v, n1k, n1v

        s0k, s0v, s1k, s1v = lax.fori_loop(
            0, nch, merge_chunk, (inf16, big16, inf16, big16))
        obuf[pl.ds(pl.multiple_of(q * K, 16), 16)] = s0v + b * N
        obuf[pl.ds(pl.multiple_of(q * K + 16, 16), 16)] = s1v + b * N
        return 0

    lax.fori_loop(0, QPW, do_row, 0)
    pltpu.sync_copy(obuf, out_hbm.at[w])


def _knn_sc(xyz_t, fps_idx):
    mesh = plsc.VectorSubcoreMesh(core_axis_name="c", subcore_axis_name="s")
    return pl.kernel(
        _knn_sc_body,
        mesh=mesh,
        compiler_params=pltpu.CompilerParams(needs_layout_passes=False),
        out_type=jax.ShapeDtypeStruct((NW, RPW), jnp.int32),
        scratch_types=[
            pltpu.VMEM((3, N + 16), jnp.float32),
            pltpu.VMEM((NPOINT,), jnp.int32),
            pltpu.VMEM((N + 16,), jnp.float32),
            pltpu.VMEM((N + 16,), jnp.int32),
            pltpu.VMEM((RPW,), jnp.int32),
        ],
    )(xyz_t, fps_idx)


_GCH = 128
_NCH = RPW // _GCH


def _gather_sc_body(p1_hbm, idx_hbm, out_hbm, idx_v, buf0, buf1, sem0, sem1):
    w = lax.axis_index("c") * 16 + lax.axis_index("s")
    pltpu.sync_copy(idx_hbm.at[w], idx_v)
    bufs = (buf0, buf1)
    sems = (sem0, sem1)
    handles = [
        pltpu.async_copy(p1_hbm.at[idx_v.at[j]], bufs[j], sems[j])
        for j in range(2)
    ]
    base = w * RPW
    for j in range(_NCH):
        slot = j % 2
        handles[slot].wait()
        pltpu.sync_copy(bufs[slot], out_hbm.at[pl.ds(base + j * _GCH, _GCH)])
        if j + 2 < _NCH:
            handles[slot] = pltpu.async_copy(
                p1_hbm.at[idx_v.at[j + 2]], bufs[slot], sems[slot])


def _gather_sc(p1_flat, gidx):
    mesh = plsc.VectorSubcoreMesh(core_axis_name="c", subcore_axis_name="s")
    return pl.kernel(
        _gather_sc_body,
        mesh=mesh,
        compiler_params=pltpu.CompilerParams(needs_layout_passes=False),
        out_type=jax.ShapeDtypeStruct((M, 128), jnp.float32),
        scratch_types=[
            pltpu.VMEM((_NCH, _GCH), jnp.int32),
            pltpu.VMEM((_GCH, 128), jnp.float32),
            pltpu.VMEM((_GCH, 128), jnp.float32),
            pltpu.SemaphoreType.DMA,
            pltpu.SemaphoreType.DMA,
        ],
    )(p1_flat, gidx.reshape(NW, _NCH, _GCH))


def _p1q1_body(xyz_ref, feat_ref, nxyz_ref, a_ref, f_ref, p1_ref, q1_ref):
    x = xyz_ref[0]
    p = lax.dot_general(x, a_ref[...], (((1,), (1,)), ((), ())),
                        preferred_element_type=jnp.float32)
    p = p + lax.dot_general(feat_ref[0], f_ref[...], (((0,), (1,)), ((), ())),
                            preferred_element_type=jnp.float32)
    p1_ref[0] = jnp.concatenate([p, jnp.zeros_like(p)], axis=1)
    q1_ref[0] = lax.dot_general(nxyz_ref[0], a_ref[...], (((1,), (1,)), ((), ())),
                                preferred_element_type=jnp.float32)


def _p1q1(xyz, features, new_xyz, W1):
    A1 = W1[:, :3]
    F1 = W1[:, 3:]
    return pl.pallas_call(
        _p1q1_body,
        grid=(B,),
        in_specs=[pl.BlockSpec((1, N, 3), lambda i: (i, 0, 0)),
                  pl.BlockSpec((1, 64, N), lambda i: (i, 0, 0)),
                  pl.BlockSpec((1, NPOINT, 3), lambda i: (i, 0, 0)),
                  pl.BlockSpec((64, 3), lambda i: (0, 0)),
                  pl.BlockSpec((64, 64), lambda i: (0, 0))],
        out_specs=[pl.BlockSpec((1, N, 128), lambda i: (i, 0, 0)),
                   pl.BlockSpec((1, NPOINT, 64), lambda i: (i, 0, 0))],
        out_shape=[jax.ShapeDtypeStruct((B, N, 128), jnp.float32),
                   jax.ShapeDtypeStruct((B, NPOINT, 64), jnp.float32)],
    )(xyz, features, new_xyz, A1, F1)


def _stats1_body(g_ref, q_ref, stats_ref, acc_ref):
    i = pl.program_id(0)
    y = g_ref[...][:, :64].reshape(QB, K, 64) - q_ref[...][:, None, :]

    @pl.when(i == 0)
    def _():
        acc_ref[...] = jnp.zeros_like(acc_ref)

    acc_ref[0, :] += jnp.sum(y, axis=(0, 1))
    acc_ref[1, :] += jnp.sum(y * y, axis=(0, 1))

    @pl.when(i == pl.num_programs(0) - 1)
    def _():
        stats_ref[...] = acc_ref[...]


def _l2_body(g_ref, q_ref, stats_ref, g1_ref, b1_ref, w_ref, o_ref, ostats_ref,
             acc_ref):
    i = pl.program_id(0)
    mean = stats_ref[0, :] * (1.0 / M)
    var = stats_ref[1, :] * (1.0 / M) - mean * mean
    a = g1_ref[0, :] * lax.rsqrt(var + EPS)
    c = b1_ref[0, :] - mean * a
    y1 = g_ref[...][:, :64].reshape(QB, K, 64) - q_ref[...][:, None, :]
    x = jnp.maximum(y1.reshape(MB, 64) * a[None, :] + c[None, :], 0.0)
    y = lax.dot_general(x, w_ref[...], (((1,), (1,)), ((), ())),
                        preferred_element_type=jnp.float32)
    o_ref[...] = y

    @pl.when(i == 0)
    def _():
        acc_ref[...] = jnp.zeros_like(acc_ref)

    acc_ref[0, :] += jnp.sum(y, axis=0)
    acc_ref[1, :] += jnp.sum(y * y, axis=0)

    @pl.when(i == pl.num_programs(0) - 1)
    def _():
        ostats_ref[...] = acc_ref[...]


def _mid_body(y_ref, stats_ref, g_ref, b_ref, w_ref, o_ref, ostats_ref, acc_ref):
    i = pl.program_id(0)
    mean = stats_ref[0, :] * (1.0 / M)
    var = stats_ref[1, :] * (1.0 / M) - mean * mean
    a = g_ref[0, :] * lax.rsqrt(var + EPS)
    c = b_ref[0, :] - mean * a
    x = jnp.maximum(y_ref[...] * a[None, :] + c[None, :], 0.0)
    y = lax.dot_general(x, w_ref[...], (((1,), (1,)), ((), ())),
                        preferred_element_type=jnp.float32)
    o_ref[...] = y

    @pl.when(i == 0)
    def _():
        acc_ref[...] = jnp.zeros_like(acc_ref)

    acc_ref[0, :] += jnp.sum(y, axis=0)
    acc_ref[1, :] += jnp.sum(y * y, axis=0)

    @pl.when(i == pl.num_programs(0) - 1)
    def _():
        ostats_ref[...] = acc_ref[...]


def _final_body(y_ref, stats_ref, g_ref, b_ref, o_ref):
    mean = stats_ref[0, :] * (1.0 / M)
    var = stats_ref[1, :] * (1.0 / M) - mean * mean
    a = g_ref[0, :] * lax.rsqrt(var + EPS)
    c = b_ref[0, :] - mean * a
    x = jnp.maximum(y_ref[...] * a[None, :] + c[None, :], 0.0)
    xr = x.reshape(QB, K, x.shape[-1])
    o_ref[...] = jnp.max(xr, axis=1)


def _mlp(G, Q1, g1, b1, W2, g2, b2, W3, g3, b3):
    grid = M // MB
    f32 = jnp.float32
    s1 = pl.pallas_call(
        _stats1_body,
        grid=(grid,),
        in_specs=[pl.BlockSpec((MB, 128), lambda i: (i, 0)),
                  pl.BlockSpec((QB, 64), lambda i: (i, 0))],
        out_specs=pl.BlockSpec((2, 64), lambda i: (0, 0)),
        out_shape=jax.ShapeDtypeStruct((2, 64), f32),
        scratch_shapes=[pltpu.VMEM((2, 64), f32)],
    )(G, Q1)

    y2, s2 = pl.pallas_call(
        _l2_body,
        grid=(grid,),
        in_specs=[pl.BlockSpec((MB, 128), lambda i: (i, 0)),
                  pl.BlockSpec((QB, 64), lambda i: (i, 0)),
                  pl.BlockSpec((2, 64), lambda i: (0, 0)),
                  pl.BlockSpec((1, 64), lambda i: (0, 0)),
                  pl.BlockSpec((1, 64), lambda i: (0, 0)),
                  pl.BlockSpec((64, 64), lambda i: (0, 0))],
        out_specs=[pl.BlockSpec((MB, 64), lambda i: (i, 0)),
                   pl.BlockSpec((2, 64), lambda i: (0, 0))],
        out_shape=[jax.ShapeDtypeStruct((M, 64), f32),
                   jax.ShapeDtypeStruct((2, 64), f32)],
        scratch_shapes=[pltpu.VMEM((2, 64), f32)],
    )(G, Q1, s1, g1.reshape(1, -1), b1.reshape(1, -1), W2)

    y3, s3 = pl.pallas_call(
        _mid_body,
        grid=(grid,),
        in_specs=[pl.BlockSpec((MB, 64), lambda i: (i, 0)),
                  pl.BlockSpec((2, 64), lambda i: (0, 0)),
                  pl.BlockSpec((1, 64), lambda i: (0, 0)),
                  pl.BlockSpec((1, 64), lambda i: (0, 0)),
                  pl.BlockSpec((128, 64), lambda i: (0, 0))],
        out_specs=[pl.BlockSpec((MB, 128), lambda i: (i, 0)),
                   pl.BlockSpec((2, 128), lambda i: (0, 0))],
        out_shape=[jax.ShapeDtypeStruct((M, 128), f32),
                   jax.ShapeDtypeStruct((2, 128), f32)],
        scratch_shapes=[pltpu.VMEM((2, 128), f32)],
    )(y2, s2, g2.reshape(1, -1), b2.reshape(1, -1), W3)

    out = pl.pallas_call(
        _final_body,
        grid=(grid,),
        in_specs=[pl.BlockSpec((MB, 128), lambda i: (i, 0)),
                  pl.BlockSpec((2, 128), lambda i: (0, 0)),
                  pl.BlockSpec((1, 128), lambda i: (0, 0)),
                  pl.BlockSpec((1, 128), lambda i: (0, 0))],
        out_specs=pl.BlockSpec((QB, 128), lambda i: (i, 0)),
        out_shape=jax.ShapeDtypeStruct((M // K, 128), f32),
    )(y3, s3, g3.reshape(1, -1), b3.reshape(1, -1))
    return out


def kernel(xyz, features, W1, g1, b1, W2, g2, b2, W3, g3, b3):
    far0 = jax.random.randint(jax.random.key(1), (B,), 0, N).astype(jnp.int32)
    far0_pad = jnp.pad(far0, (0, 32 - B))
    xyz_t_pad = jnp.pad(jnp.transpose(xyz, (0, 2, 1)), ((0, 0), (0, 0), (0, 16)))
    fps_idx = _fps_sc(xyz_t_pad, far0_pad)
    new_xyz = _idx_points(xyz, fps_idx)
    gidx = _knn_sc(xyz_t_pad, fps_idx)
    P1, Q1 = _p1q1(xyz, features, new_xyz, W1)
    G = _gather_sc(P1.reshape(B * N, 128), gidx.reshape(M))
    out = _mlp(G, Q1.reshape(B * NPOINT, 64), g1, b1, W2, g2, b2, W3, g3, b3)
    new_features = jnp.transpose(out.reshape(B, NPOINT, 128), (0, 2, 1))
    return (new_xyz, new_features)


# speedup vs baseline: 1.2818x; 1.2818x over previous
import functools

import jax
import jax.numpy as jnp
from jax import lax
from jax.experimental import pallas as pl
from jax.experimental.pallas import tpu as pltpu
from jax.experimental.pallas import tpu_sc as plsc

NPOINT = 512
K = 32
B = 16
N = 2048
M = B * NPOINT * K
MB = 4096
QB = MB // K
EPS = 1e-5
NW = 32
RPW = M // NW
QPW = B * NPOINT // NW


def _idx_points(points, idx):
    return jax.vmap(lambda p, i: p[i])(points, idx)


def _fps_sc_body(xyzd_hbm, far0_hbm, out_hbm, xyzd_v, cent_v, far_v):
    wid = lax.axis_index("c") * 16 + lax.axis_index("s")

    @pl.when(wid < B)
    def _():
        pltpu.sync_copy(xyzd_hbm.at[wid], xyzd_v)
        pltpu.sync_copy(far0_hbm, far_v)
        nchunks = N // 16
        lanes = lax.iota(jnp.int32, 16)

        row0 = jnp.zeros((16,), jnp.int32)
        row1 = jnp.ones((16,), jnp.int32)
        row2 = jnp.full((16,), 2, jnp.int32)

        def step(i, carry):
            far_b, cvec = carry
            cvec = jnp.where(lanes == (i % 16), far_b, cvec)

            @pl.when((i % 16) == 15)
            def _():
                cent_v[pl.ds(pl.multiple_of(i - 15, 16), 16)] = cvec

            fhi = lax.shift_right_logical(far_b, 4)
            flo = lax.bitwise_and(far_b, 15)
            cx = plsc.load_gather(xyzd_v, [fhi, row0, flo])
            cy = plsc.load_gather(xyzd_v, [fhi, row1, flo])
            cz = plsc.load_gather(xyzd_v, [fhi, row2, flo])

            def chunk4(c4, carry):
                out = []
                for u in range(4):
                    bv, bi = carry[2 * u], carry[2 * u + 1]
                    c = c4 * 4 + u
                    dx = xyzd_v[c, 0, :] - cx
                    dy = xyzd_v[c, 1, :] - cy
                    dz = xyzd_v[c, 2, :] - cz
                    d = dx * dx + dy * dy + dz * dz
                    dmin = jnp.minimum(xyzd_v[c, 3, :], d)
                    xyzd_v[c, 3, :] = dmin
                    m = dmin > bv
                    out.append(jnp.where(m, dmin, bv))
                    out.append(jnp.where(m, c * 16 + lanes, bi))
                return tuple(out)

            bv0 = jnp.full((16,), -1.0, jnp.float32)
            bi0 = jnp.zeros((16,), jnp.int32)
            acc = lax.fori_loop(0, nchunks // 4, chunk4,
                                (bv0, bi0) * 4, unroll=2)

            def comb(a, b):
                av, ai = a
                bv_, bi_ = b
                m = (bv_ > av) | ((bv_ == av) & (bi_ < ai))
                return jnp.where(m, bv_, av), jnp.where(m, bi_, ai)

            bv, bi = comb(comb((acc[0], acc[1]), (acc[2], acc[3])),
                          comb((acc[4], acc[5]), (acc[6], acc[7])))
            maxv = jnp.max(bv)
            cand = jnp.where(bv == maxv, bi, jnp.int32(1 << 30))
            far_new = jnp.full((16,), jnp.min(cand), jnp.int32)
            return far_new, cvec

        far_init = plsc.load_gather(far_v, [jnp.full((16,), wid, jnp.int32)])
        lax.fori_loop(0, NPOINT, step, (far_init, jnp.zeros((16,), jnp.int32)))
        pltpu.sync_copy(cent_v, out_hbm.at[wid])


def _fps_sc(xyzd, far0):
    mesh = plsc.VectorSubcoreMesh(core_axis_name="c", subcore_axis_name="s")
    return pl.kernel(
        _fps_sc_body,
        mesh=mesh,
        compiler_params=pltpu.CompilerParams(needs_layout_passes=False),
        out_type=jax.ShapeDtypeStruct((B, NPOINT), jnp.int32),
        scratch_types=[
            pltpu.VMEM((N // 16, 4, 16), jnp.float32),
            pltpu.VMEM((NPOINT,), jnp.int32),
            pltpu.VMEM((32,), jnp.int32),
        ],
    )(xyzd, far0)


def _knn_sc_body(xyzd_hbm, fps_hbm, out_hbm, xyzd_v, fps_v, dbuf, cidx, obuf):
    w = lax.axis_index("c") * 16 + lax.axis_index("s")
    b = w // 2
    qbase = (w % 2) * QPW
    lanes = lax.iota(jnp.int32, 16)
    inf16 = jnp.full((16,), jnp.inf, jnp.float32)
    row0 = jnp.zeros((16,), jnp.int32)
    row1 = jnp.ones((16,), jnp.int32)
    row2 = jnp.full((16,), 2, jnp.int32)

    pltpu.sync_copy(xyzd_hbm.at[b], xyzd_v)
    pltpu.sync_copy(fps_hbm.at[b], fps_v)
    dbuf[pl.ds(N, 16)] = inf16

    def do_row(q, _):
        qi = qbase + q
        farb = plsc.load_gather(fps_v, [jnp.full((16,), qi, jnp.int32)])
        fhi = lax.shift_right_logical(farb, 4)
        flo = lax.bitwise_and(farb, 15)
        cx = plsc.load_gather(xyzd_v, [fhi, row0, flo])
        cy = plsc.load_gather(xyzd_v, [fhi, row1, flo])
        cz = plsc.load_gather(xyzd_v, [fhi, row2, flo])

        def dchunk4(c4, hs):
            out = []
            for u in range(4):
                c = c4 * 4 + u
                dx = xyzd_v[c, 0, :] - cx
                dy = xyzd_v[c, 1, :] - cy
                dz = xyzd_v[c, 2, :] - cz
                d = dx * dx + dy * dy + dz * dz
                dbuf[pl.ds(c * 16, 16)] = d
                out.append(jnp.minimum(hs[u], d))
            return tuple(out)

        hs1 = lax.fori_loop(0, 16, dchunk4, (inf16,) * 4)
        hs2 = lax.fori_loop(16, 32, dchunk4, (inf16,) * 4)
        h1 = jnp.minimum(jnp.minimum(hs1[0], hs1[1]), jnp.minimum(hs1[2], hs1[3]))
        h2 = jnp.minimum(jnp.minimum(hs2[0], hs2[1]), jnp.minimum(hs2[2], hs2[3]))
        thr = jnp.maximum(jnp.max(h1), jnp.max(h2))

        def cchunk(c, off):
            base = c * 16
            d = dbuf[pl.ds(base, 16)]
            m = d <= thr
            cs = plsc.cumsum(m.astype(jnp.int32))
            pos = off + cs - 1
            plsc.store_scatter(cidx, [pos], base + lanes, mask=m)
            return off + plsc.all_reduce_population_count(m)[0]

        off = lax.fori_loop(0, 128, cchunk, jnp.int32(0), unroll=4)
        plsc.store_scatter(cidx, [off + lanes], jnp.full((16,), N, jnp.int32))
        nch = (off + 15) // 16

        big16 = jnp.full((16,), 1 << 30, jnp.int32)

        def merge_chunk(c, carry):
            s0k, s0v, s1k, s1v = carry
            idxs = cidx[pl.ds(pl.multiple_of(c * 16, 16), 16)]
            vals = plsc.load_gather(dbuf, [idxs])
            ck, cv = plsc.sort_key_val(vals, idxs)
            rk = lax.rev(ck, (0,))
            rv = lax.rev(cv, (0,))
            m = s1k <= rk
            lok = jnp.where(m, s1k, rk)
            lov = jnp.where(m, s1v, rv)
            lok, lov = plsc.sort_key_val(lok, lov)
            rk = lax.rev(lok, (0,))
            rv = lax.rev(lov, (0,))
            m = s0k <= rk
            n0k = jnp.where(m, s0k, rk)
            n0v = jnp.where(m, s0v, rv)
            n1k = jnp.where(m, rk, s0k)
            n1v = jnp.where(m, rv, s0v)
            n0k, n0v = plsc.sort_key_val(n0k, n0v)
            n1k, n1v = plsc.sort_key_val(n1k, n1v)
            return n0k, n0v, n1k, n1v

        s0k, s0v, s1k, s1v = lax.fori_loop(
            0, nch, merge_chunk, (inf16, big16, inf16, big16))
        obuf[pl.ds(pl.multiple_of(q * K, 16), 16)] = s0v + b * N
        obuf[pl.ds(pl.multiple_of(q * K + 16, 16), 16)] = s1v + b * N
        return 0

    lax.fori_loop(0, QPW, do_row, 0)
    pltpu.sync_copy(obuf, out_hbm.at[w])


def _knn_sc(xyzd, fps_idx):
    mesh = plsc.VectorSubcoreMesh(core_axis_name="c", subcore_axis_name="s")
    return pl.kernel(
        _knn_sc_body,
        mesh=mesh,
        compiler_params=pltpu.CompilerParams(needs_layout_passes=False),
        out_type=jax.ShapeDtypeStruct((NW, RPW), jnp.int32),
        scratch_types=[
            pltpu.VMEM((N // 16, 4, 16), jnp.float32),
            pltpu.VMEM((NPOINT,), jnp.int32),
            pltpu.VMEM((N + 16,), jnp.float32),
            pltpu.VMEM((N + 16,), jnp.int32),
            pltpu.VMEM((RPW,), jnp.int32),
        ],
    )(xyzd, fps_idx)


_GCH = 128
_NCH = RPW // _GCH


def _gather_sc_body(p1_hbm, idx_hbm, out_hbm, idx_v, buf0, buf1, sem0, sem1):
    w = lax.axis_index("c") * 16 + lax.axis_index("s")
    pltpu.sync_copy(idx_hbm.at[w], idx_v)
    bufs = (buf0, buf1)
    sems = (sem0, sem1)
    handles = [
        pltpu.async_copy(p1_hbm.at[idx_v.at[j]], bufs[j], sems[j])
        for j in range(2)
    ]
    base = w * RPW
    for j in range(_NCH):
        slot = j % 2
        handles[slot].wait()
        pltpu.sync_copy(bufs[slot], out_hbm.at[pl.ds(base + j * _GCH, _GCH)])
        if j + 2 < _NCH:
            handles[slot] = pltpu.async_copy(
                p1_hbm.at[idx_v.at[j + 2]], bufs[slot], sems[slot])


def _gather_sc(p1_flat, gidx):
    mesh = plsc.VectorSubcoreMesh(core_axis_name="c", subcore_axis_name="s")
    return pl.kernel(
        _gather_sc_body,
        mesh=mesh,
        compiler_params=pltpu.CompilerParams(needs_layout_passes=False),
        out_type=jax.ShapeDtypeStruct((M, 128), jnp.float32),
        scratch_types=[
            pltpu.VMEM((_NCH, _GCH), jnp.int32),
            pltpu.VMEM((_GCH, 128), jnp.float32),
            pltpu.VMEM((_GCH, 128), jnp.float32),
            pltpu.SemaphoreType.DMA,
            pltpu.SemaphoreType.DMA,
        ],
    )(p1_flat, gidx.reshape(NW, _NCH, _GCH))


def _p1q1_body(xyz_ref, feat_ref, nxyz_ref, a_ref, f_ref, p1_ref, q1_ref):
    x = xyz_ref[0]
    p = lax.dot_general(x, a_ref[...], (((1,), (1,)), ((), ())),
                        preferred_element_type=jnp.float32)
    p = p + lax.dot_general(feat_ref[0], f_ref[...], (((0,), (1,)), ((), ())),
                            preferred_element_type=jnp.float32)
    p1_ref[0] = jnp.concatenate([p, jnp.zeros_like(p)], axis=1)
    q1_ref[0] = lax.dot_general(nxyz_ref[0], a_ref[...], (((1,), (1,)), ((), ())),
                                preferred_element_type=jnp.float32)


def _p1q1(xyz, features, new_xyz, W1):
    A1 = W1[:, :3]
    F1 = W1[:, 3:]
    return pl.pallas_call(
        _p1q1_body,
        grid=(B,),
        in_specs=[pl.BlockSpec((1, N, 3), lambda i: (i, 0, 0)),
                  pl.BlockSpec((1, 64, N), lambda i: (i, 0, 0)),
                  pl.BlockSpec((1, NPOINT, 3), lambda i: (i, 0, 0)),
                  pl.BlockSpec((64, 3), lambda i: (0, 0)),
                  pl.BlockSpec((64, 64), lambda i: (0, 0))],
        out_specs=[pl.BlockSpec((1, N, 128), lambda i: (i, 0, 0)),
                   pl.BlockSpec((1, NPOINT, 64), lambda i: (i, 0, 0))],
        out_shape=[jax.ShapeDtypeStruct((B, N, 128), jnp.float32),
                   jax.ShapeDtypeStruct((B, NPOINT, 64), jnp.float32)],
    )(xyz, features, new_xyz, A1, F1)


def _stats1_body(g_ref, q_ref, stats_ref, acc_ref):
    i = pl.program_id(0)
    y = g_ref[...][:, :64].reshape(QB, K, 64) - q_ref[...][:, None, :]

    @pl.when(i == 0)
    def _():
        acc_ref[...] = jnp.zeros_like(acc_ref)

    acc_ref[0, :] += jnp.sum(y, axis=(0, 1))
    acc_ref[1, :] += jnp.sum(y * y, axis=(0, 1))

    @pl.when(i == pl.num_programs(0) - 1)
    def _():
        stats_ref[...] = acc_ref[...]


def _l2_body(g_ref, q_ref, stats_ref, g1_ref, b1_ref, w_ref, o_ref, ostats_ref,
             acc_ref):
    i = pl.program_id(0)
    mean = stats_ref[0, :] * (1.0 / M)
    var = stats_ref[1, :] * (1.0 / M) - mean * mean
    a = g1_ref[0, :] * lax.rsqrt(var + EPS)
    c = b1_ref[0, :] - mean * a
    y1 = g_ref[...][:, :64].reshape(QB, K, 64) - q_ref[...][:, None, :]
    x = jnp.maximum(y1.reshape(MB, 64) * a[None, :] + c[None, :], 0.0)
    y = lax.dot_general(x, w_ref[...], (((1,), (1,)), ((), ())),
                        preferred_element_type=jnp.float32)
    o_ref[...] = y

    @pl.when(i == 0)
    def _():
        acc_ref[...] = jnp.zeros_like(acc_ref)

    acc_ref[0, :] += jnp.sum(y, axis=0)
    acc_ref[1, :] += jnp.sum(y * y, axis=0)

    @pl.when(i == pl.num_programs(0) - 1)
    def _():
        ostats_ref[...] = acc_ref[...]


def _mid_body(y_ref, stats_ref, g_ref, b_ref, w_ref, o_ref, ostats_ref, acc_ref):
    i = pl.program_id(0)
    mean = stats_ref[0, :] * (1.0 / M)
    var = stats_ref[1, :] * (1.0 / M) - mean * mean
    a = g_ref[0, :] * lax.rsqrt(var + EPS)
    c = b_ref[0, :] - mean * a
    x = jnp.maximum(y_ref[...] * a[None, :] + c[None, :], 0.0)
    y = lax.dot_general(x, w_ref[...], (((1,), (1,)), ((), ())),
                        preferred_element_type=jnp.float32)
    o_ref[...] = y

    @pl.when(i == 0)
    def _():
        acc_ref[...] = jnp.zeros_like(acc_ref)

    acc_ref[0, :] += jnp.sum(y, axis=0)
    acc_ref[1, :] += jnp.sum(y * y, axis=0)

    @pl.when(i == pl.num_programs(0) - 1)
    def _():
        ostats_ref[...] = acc_ref[...]


def _final_body(y_ref, stats_ref, g_ref, b_ref, o_ref):
    mean = stats_ref[0, :] * (1.0 / M)
    var = stats_ref[1, :] * (1.0 / M) - mean * mean
    a = g_ref[0, :] * lax.rsqrt(var + EPS)
    c = b_ref[0, :] - mean * a
    x = jnp.maximum(y_ref[...] * a[None, :] + c[None, :], 0.0)
    xr = x.reshape(QB, K, x.shape[-1])
    o_ref[...] = jnp.max(xr, axis=1)


def _mlp(G, Q1, g1, b1, W2, g2, b2, W3, g3, b3):
    grid = M // MB
    f32 = jnp.float32
    s1 = pl.pallas_call(
        _stats1_body,
        grid=(grid,),
        in_specs=[pl.BlockSpec((MB, 128), lambda i: (i, 0)),
                  pl.BlockSpec((QB, 64), lambda i: (i, 0))],
        out_specs=pl.BlockSpec((2, 64), lambda i: (0, 0)),
        out_shape=jax.ShapeDtypeStruct((2, 64), f32),
        scratch_shapes=[pltpu.VMEM((2, 64), f32)],
    )(G, Q1)

    y2, s2 = pl.pallas_call(
        _l2_body,
        grid=(grid,),
        in_specs=[pl.BlockSpec((MB, 128), lambda i: (i, 0)),
                  pl.BlockSpec((QB, 64), lambda i: (i, 0)),
                  pl.BlockSpec((2, 64), lambda i: (0, 0)),
                  pl.BlockSpec((1, 64), lambda i: (0, 0)),
                  pl.BlockSpec((1, 64), lambda i: (0, 0)),
                  pl.BlockSpec((64, 64), lambda i: (0, 0))],
        out_specs=[pl.BlockSpec((MB, 64), lambda i: (i, 0)),
                   pl.BlockSpec((2, 64), lambda i: (0, 0))],
        out_shape=[jax.ShapeDtypeStruct((M, 64), f32),
                   jax.ShapeDtypeStruct((2, 64), f32)],
        scratch_shapes=[pltpu.VMEM((2, 64), f32)],
    )(G, Q1, s1, g1.reshape(1, -1), b1.reshape(1, -1), W2)

    y3, s3 = pl.pallas_call(
        _mid_body,
        grid=(grid,),
        in_specs=[pl.BlockSpec((MB, 64), lambda i: (i, 0)),
                  pl.BlockSpec((2, 64), lambda i: (0, 0)),
                  pl.BlockSpec((1, 64), lambda i: (0, 0)),
                  pl.BlockSpec((1, 64), lambda i: (0, 0)),
                  pl.BlockSpec((128, 64), lambda i: (0, 0))],
        out_specs=[pl.BlockSpec((MB, 128), lambda i: (i, 0)),
                   pl.BlockSpec((2, 128), lambda i: (0, 0))],
        out_shape=[jax.ShapeDtypeStruct((M, 128), f32),
                   jax.ShapeDtypeStruct((2, 128), f32)],
        scratch_shapes=[pltpu.VMEM((2, 128), f32)],
    )(y2, s2, g2.reshape(1, -1), b2.reshape(1, -1), W3)

    out = pl.pallas_call(
        _final_body,
        grid=(grid,),
        in_specs=[pl.BlockSpec((MB, 128), lambda i: (i, 0)),
                  pl.BlockSpec((2, 128), lambda i: (0, 0)),
                  pl.BlockSpec((1, 128), lambda i: (0, 0)),
                  pl.BlockSpec((1, 128), lambda i: (0, 0))],
        out_specs=pl.BlockSpec((QB, 128), lambda i: (i, 0)),
        out_shape=jax.ShapeDtypeStruct((M // K, 128), f32),
    )(y3, s3, g3.reshape(1, -1), b3.reshape(1, -1))
    return out


def kernel(xyz, features, W1, g1, b1, W2, g2, b2, W3, g3, b3):
    far0 = jax.random.randint(jax.random.key(1), (B,), 0, N).astype(jnp.int32)
    far0_pad = jnp.pad(far0, (0, 32 - B))
    xyz_c = jnp.transpose(xyz.reshape(B, N // 16, 16, 3), (0, 1, 3, 2))
    xyzd = jnp.concatenate(
        [xyz_c, jnp.full((B, N // 16, 1, 16), 1e10, jnp.float32)], axis=2)
    fps_idx = _fps_sc(xyzd, far0_pad)
    new_xyz = _idx_points(xyz, fps_idx)
    gidx = _knn_sc(xyzd, fps_idx)
    P1, Q1 = _p1q1(xyz, features, new_xyz, W1)
    G = _gather_sc(P1.reshape(B * N, 128), gidx.reshape(M))
    out = _mlp(G, Q1.reshape(B * NPOINT, 64), g1, b1, W2, g2, b2, W3, g3, b3)
    new_features = jnp.transpose(out.reshape(B, NPOINT, 128), (0, 2, 1))
    return (new_xyz, new_features)
